# Initial kernel scaffold; baseline (speedup 1.0000x reference)
import jax
import jax.numpy as jnp
from jax.experimental import pallas as pl


def kernel(logits, targets, class_weights):
    raise NotImplementedError("write your pallas kernel here")


# speedup vs baseline: 32.8794x; 32.8794x over previous
import functools

import numpy as np
import jax
import jax.numpy as jnp
from jax import lax
from jax.experimental import pallas as pl
from jax.experimental.pallas import tpu as pltpu
from jax.experimental.pallas import tpu_sc as plsc

_LN = 16
_KB = 2048
_SH = 19
_REMMASK = (1 << _SH) - 1
_BH = 64
_CW = 8192


def _pack_body(cw_ref, logits_ref, targets_ref, words_ref, acc_ref):
    C = logits_ref.shape[1]
    x = logits_ref[0]
    t = targets_ref[0]
    m = jnp.max(x, axis=0)
    ex = jnp.exp(x - m[None])
    s = jnp.sum(ex, axis=0)
    inv_s = 1.0 / s
    logs = jnp.log(s)
    xt = jnp.zeros_like(m)
    wmap = jnp.zeros_like(m)
    lane = lax.broadcasted_iota(jnp.int32, (1, 1, 128), 2)
    row = jnp.zeros((1, 1, 128), jnp.float32)
    for c in range(C):
        fgm = t == c
        p = ex[c] * inv_s
        err = jnp.where(fgm, 1.0 - p, p)
        uf = jnp.clip(err * np.float32(2.0 ** 30), 0.0,
                      np.float32(2 ** 30 - 64))
        words_ref[c, 0] = uf.astype(jnp.int32)
        fgf = fgm.astype(jnp.float32)
        xt = xt + x[c] * fgf
        wmap = wmap + cw_ref[c] * fgf
        row = row + jnp.sum(fgf) * (lane == (2 + c)).astype(jnp.float32)
    nll = logs + m - xt
    row = row + jnp.sum(wmap * nll) * (lane == 0).astype(jnp.float32)
    row = row + jnp.sum(wmap) * (lane == 1).astype(jnp.float32)
    acc_ref[...] = row


def _make_sc_hist(M, n_log2, nc, ns):
    nw = nc * ns
    chunk = M // nw
    assert chunk % _CW == 0 and (1 << n_log2) % _CW == 0
    num_slots = 2 * nw
    mesh = plsc.VectorSubcoreMesh(core_axis_name="c", subcore_axis_name="s")

    @functools.partial(
        pl.kernel,
        out_type=[
            jax.ShapeDtypeStruct((num_slots, _KB * _LN), jnp.float32),
            jax.ShapeDtypeStruct((num_slots, _KB * _LN), jnp.float32),
        ],
        mesh=mesh,
        scratch_types=[
            pltpu.VMEM((_CW,), jnp.int32),
            pltpu.VMEM((_KB * _LN,), jnp.float32),
            pltpu.VMEM((_KB * _LN,), jnp.float32),
        ],
        compiler_params=pltpu.CompilerParams(needs_layout_passes=False),
    )
    def sc_hist(words, cnt_out, rem_out, buf, cnt, rsum):
        wid = lax.axis_index("s") * nc + lax.axis_index("c")
        g0 = wid * chunk
        gend = g0 + chunk
        c0 = lax.shift_right_logical(g0, n_log2)
        end_a = jnp.minimum(gend, lax.shift_left(c0 + 1, n_log2))
        lane_off = jnp.arange(16, dtype=jnp.int32) * _KB
        ones = jnp.full((16,), 1.0, jnp.float32)
        zf = jnp.zeros((16,), jnp.float32)

        for seg in range(2):
            start = g0 if seg == 0 else end_a
            end = end_a if seg == 0 else gend

            def zbody(i, _):
                cnt[pl.ds(i * 16, 16)] = zf
                rsum[pl.ds(i * 16, 16)] = zf
                return 0

            lax.fori_loop(0, (_KB * _LN) // 16, zbody, 0)

            nch = lax.shift_right_logical(end - start, 13)

            def chbody(ch, _):
                off = pl.multiple_of(start + ch * _CW, _CW)
                pltpu.sync_copy(words.at[pl.ds(off, _CW)], buf)

                def vbody(v, _):
                    for uu in range(4):
                        wv = buf[pl.ds((v * 4 + uu) * 16, 16)]
                        bi = lax.shift_right_logical(wv, _SH) + lane_off
                        rem = (wv & _REMMASK).astype(jnp.float32)
                        plsc.addupdate_scatter(cnt, [bi], ones)
                        plsc.addupdate_scatter(rsum, [bi], rem)
                    return 0

                lax.fori_loop(0, _CW // 64, vbody, 0)
                return 0

            lax.fori_loop(0, nch, chbody, 0)
            slot = 2 * wid + seg
            pltpu.sync_copy(cnt, cnt_out.at[slot])
            pltpu.sync_copy(rsum, rem_out.at[slot])

    return sc_hist


def _make_final_body(slot_classes, N, C):
    HIGH = lax.Precision.HIGHEST

    def _final_body(cnt_ref, rem_ref, acc_ref, tot_ref, wce_ref, lov_ref):
        accv = acc_ref[...]
        acc2 = accv[:, 0, :]
        lane2 = lax.broadcasted_iota(jnp.int32, acc2.shape, 1)
        num = jnp.sum(jnp.where(lane2 == 0, acc2, 0.0))
        den = jnp.sum(jnp.where(lane2 == 1, acc2, 0.0))
        wce = num / den

        r128 = lax.broadcasted_iota(jnp.int32, (128, 128), 0)
        c128 = lax.broadcasted_iota(jnp.int32, (128, 128), 1)
        u_incl = (r128 <= c128).astype(jnp.float32)
        r16 = lax.broadcasted_iota(jnp.int32, (16, 16), 0)
        c16 = lax.broadcasted_iota(jnp.int32, (16, 16), 1)
        l_strict = (c16 < r16).astype(jnp.float32)
        rr = lax.broadcasted_iota(jnp.int32, (16, 128), 0)
        cc = lax.broadcasted_iota(jnp.int32, (16, 128), 1)
        binval = (rr * 128 + cc).astype(jnp.float32) * np.float32(1.0 / _KB)

        n_present = jnp.float32(0.0)
        lov_sum = jnp.float32(0.0)
        for c in range(C):
            slots = [s for s in range(len(slot_classes))
                     if slot_classes[s] == c]
            hm4 = cnt_ref[slots[0]]
            rm4 = rem_ref[slots[0]]
            for s in slots[1:]:
                hm4 = hm4 + cnt_ref[s]
                rm4 = rm4 + rem_ref[s]
            hm = jnp.sum(hm4, axis=0)
            rm = jnp.sum(rm4, axis=0)
            rowcum = jnp.dot(hm, u_incl, precision=HIGH)
            rowlast = rowcum[:, 127:128]
            offs = jnp.dot(l_strict, rowlast, precision=HIGH)
            cin = rowcum + offs
            g = np.float32(N) - cin
            sv = hm * binval + rm * np.float32(2.0 ** -30)
            corr = jnp.where(hm > 0, hm * hm - 1.0, 0.0) \
                * np.float32(1.0 / (12.0 * _KB))
            terms = sv * (g + (hm + 1.0) * 0.5) - corr
            tc = jnp.sum(jnp.sum(terms, axis=0))
            loss_c = tc * np.float32(1.0 / N)
            fg_c = jnp.sum(jnp.where(lane2 == (2 + c), acc2, 0.0))
            pres = (fg_c > 0).astype(jnp.float32)
            n_present = n_present + pres
            lov_sum = lov_sum + loss_c * pres
        lovasz = jnp.where(n_present > 0,
                           lov_sum / jnp.maximum(n_present, 1.0), 0.0)
        total = 0.5 * wce + 0.5 * lovasz
        tot_ref[...] = jnp.full((8, 128), total, jnp.float32)
        wce_ref[...] = jnp.full((8, 128), wce, jnp.float32)
        lov_ref[...] = jnp.full((8, 128), lovasz, jnp.float32)

    return _final_body


def kernel(logits, targets, class_weights):
    B, C, H, W = logits.shape
    N = B * H * W
    n_log2 = int(N).bit_length() - 1
    assert (1 << n_log2) == N
    M = C * N
    steps = B * (H // _BH)

    words, acc = pl.pallas_call(
        _pack_body,
        grid=(B, H // _BH),
        in_specs=[
            pl.BlockSpec(memory_space=pltpu.SMEM),
            pl.BlockSpec((1, C, _BH, W), lambda b, r: (b, 0, r, 0)),
            pl.BlockSpec((1, _BH, W), lambda b, r: (b, r, 0)),
        ],
        out_specs=[
            pl.BlockSpec((C, 1, _BH, W), lambda b, r: (0, b, r, 0)),
            pl.BlockSpec((1, 1, 128), lambda b, r: (b * (H // _BH) + r, 0, 0)),
        ],
        out_shape=[
            jax.ShapeDtypeStruct((C, B, H, W), jnp.int32),
            jax.ShapeDtypeStruct((steps, 1, 128), jnp.float32),
        ],
    )(class_weights, logits, targets)

    try:
        info = plsc.get_sparse_core_info()
        nc, ns = info.num_cores, info.num_subcores
    except Exception:
        nc, ns = 2, 16
    nw = nc * ns
    chunk = M // nw

    sc_hist = _make_sc_hist(M, n_log2, nc, ns)
    cnt_part, rem_part = sc_hist(words.reshape(M))

    slot_classes = []
    for s in range(2 * nw):
        wid, seg = divmod(s, 2)
        c0 = (wid * chunk) >> n_log2
        slot_classes.append(c0 if seg == 0 else min(c0 + 1, C - 1))

    shaped = (2 * nw, _LN, _KB // 128, 128)
    tot, wce, lov = pl.pallas_call(
        _make_final_body(slot_classes, N, C),
        in_specs=[
            pl.BlockSpec(shaped, lambda: (0, 0, 0, 0)),
            pl.BlockSpec(shaped, lambda: (0, 0, 0, 0)),
            pl.BlockSpec((steps, 1, 128), lambda: (0, 0, 0)),
        ],
        out_specs=[
            pl.BlockSpec((8, 128), lambda: (0, 0)),
            pl.BlockSpec((8, 128), lambda: (0, 0)),
            pl.BlockSpec((8, 128), lambda: (0, 0)),
        ],
        out_shape=[
            jax.ShapeDtypeStruct((8, 128), jnp.float32),
            jax.ShapeDtypeStruct((8, 128), jnp.float32),
            jax.ShapeDtypeStruct((8, 128), jnp.float32),
        ],
    )(cnt_part.reshape(shaped), rem_part.reshape(shaped), acc)

    return (tot[0, 0], wce[0, 0], lov[0, 0])
